# baseline (speedup 1.0000x reference)
import functools

import jax
import jax.numpy as jnp
from jax import lax
from jax.experimental import pallas as pl
from jax.experimental.pallas import tpu as pltpu
from jax.experimental.pallas import tpu_sc as plsc

BATCH = 16384
NG = 10
NJ = 5
W = 32

_INFO = plsc.get_sparse_core_info()
_NC, _NS = _INFO.num_cores, _INFO.num_subcores
_NW = _NC * _NS
_ROWS = BATCH // _NW


def _body(in_hbm, *args):
    out_hbms, sem = args[:NG], args[NG]
    wid = lax.axis_index("s") * _NC + lax.axis_index("c")
    base = wid * _ROWS
    copies = []
    for g in range(NG):
        for j in range(NJ):
            src_col = (j * NG + g) * W
            copies.append(pltpu.async_copy(
                in_hbm.at[pl.ds(base, _ROWS), pl.ds(src_col, W)],
                out_hbms[g].at[pl.ds(base, _ROWS), pl.ds(j * W, W)],
                sem,
            ))
    for c in copies:
        c.wait()


@jax.jit
def kernel(input_tensor):
    mesh = plsc.VectorSubcoreMesh(core_axis_name="c", subcore_axis_name="s")
    out_type = tuple(
        jax.ShapeDtypeStruct((BATCH, NJ * W), jnp.float32) for _ in range(NG)
    )
    return pl.kernel(
        _body,
        out_type=out_type,
        mesh=mesh,
        scratch_types=[pltpu.SemaphoreType.DMA],
        compiler_params=pltpu.CompilerParams(use_tc_tiling_on_sc=False),
    )(input_tensor)


# speedup vs baseline: 6.9615x; 6.9615x over previous
import functools

import numpy as np
import jax
import jax.numpy as jnp
from jax import lax
from jax.experimental import pallas as pl
from jax.experimental.pallas import tpu as pltpu
from jax.experimental.pallas import tpu_sc as plsc

BATCH = 16384
NG = 10
NJ = 5
W = 32
NCHUNK = 100

_INFO = plsc.get_sparse_core_info()
_NC, _NS = _INFO.num_cores, _INFO.num_subcores
_NW = _NC * _NS
_ROWS = BATCH // _NW
_R = 128
_NITER = _ROWS // _R

_IDX_NP = (
    np.arange(BATCH, dtype=np.int32)[None, :, None] * NCHUNK
    + np.arange(NJ, dtype=np.int32)[None, None, :] * NG
    + np.arange(NG, dtype=np.int32)[:, None, None]
)


def _body(in_hbm, idx_hbm, *args):
    out_hbms = args[:NG]
    idx_v = args[NG]
    bufs = args[NG + 1:NG + 3]
    isem = args[NG + 3]
    gsems = args[NG + 4]
    wsems = args[NG + 5]
    wid = lax.axis_index("s") * _NC + lax.axis_index("c")
    base = wid * _ROWS
    pltpu.async_copy(idx_hbm.at[:, pl.ds(base * NJ, _ROWS * NJ)], idx_v, isem).wait()

    @pl.loop(0, _NITER)
    def _chunk(i):
        r0 = base + i * _R
        prev = None
        wh = [None, None]
        for g in range(NG):
            p = g % 2
            if wh[p] is not None:
                wh[p].wait()
            h = pltpu.async_copy(
                in_hbm.at[idx_v.at[g, pl.ds(i * _R * NJ, _R * NJ)]],
                bufs[p],
                gsems[p],
            )
            if prev is not None:
                pg, pp, ph = prev
                ph.wait()
                wh[pp] = pltpu.async_copy(
                    bufs[pp],
                    out_hbms[pg].at[pl.ds(r0 * NJ, _R * NJ), :],
                    wsems[pp],
                )
            prev = (g, p, h)
        pg, pp, ph = prev
        ph.wait()
        wh[pp] = pltpu.async_copy(
            bufs[pp],
            out_hbms[pg].at[pl.ds(r0 * NJ, _R * NJ), :],
            wsems[pp],
        )
        for p in range(2):
            if wh[p] is not None:
                wh[p].wait()


@jax.jit
def kernel(input_tensor):
    mesh = plsc.VectorSubcoreMesh(core_axis_name="c", subcore_axis_name="s")
    out_type = tuple(
        jax.ShapeDtypeStruct((BATCH * NJ, W), jnp.float32) for _ in range(NG)
    )
    in_rows = input_tensor.reshape(BATCH * NCHUNK, W)
    idx = jnp.asarray(_IDX_NP.reshape(NG, BATCH * NJ))
    outs = pl.kernel(
        _body,
        out_type=out_type,
        mesh=mesh,
        scratch_types=[
            pltpu.VMEM((NG, _ROWS * NJ), jnp.int32),
            pltpu.VMEM((_R * NJ, W), jnp.float32),
            pltpu.VMEM((_R * NJ, W), jnp.float32),
            pltpu.SemaphoreType.DMA,
            (pltpu.SemaphoreType.DMA, pltpu.SemaphoreType.DMA),
            (pltpu.SemaphoreType.DMA, pltpu.SemaphoreType.DMA),
        ],
        compiler_params=pltpu.CompilerParams(use_tc_tiling_on_sc=False),
    )(in_rows, idx)
    return tuple(o.reshape(BATCH, NJ * W) for o in outs)
